# Initial kernel scaffold; baseline (speedup 1.0000x reference)
import jax
import jax.numpy as jnp
from jax.experimental import pallas as pl


def kernel(x, edge_index, W1, b1, W2, b2, W3, b3, W4, b4):
    raise NotImplementedError("write your pallas kernel here")


# speedup vs baseline: 11.0153x; 11.0153x over previous
import functools

import jax
import jax.numpy as jnp
from jax import lax
from jax.experimental import pallas as pl
from jax.experimental.pallas import tpu as pltpu
from jax.experimental.pallas import tpu_sc as plsc

N = 10000
E = 320000
F = 128
NP = 10240
EB = E // 128
EBP = 2560
NC, NS = 2, 16
NW = NC * NS
RPT = NP // NS

_MESH = dict(core_axis_name="c", subcore_axis_name="s", num_cores=NC,
             num_subcores=NS)


def _wid():
    return lax.axis_index("s") * NC + lax.axis_index("c")


def _blk_range(w):
    lo = ((w * EB) // NW) // 8 * 8
    hi = jnp.where(w == NW - 1, EB, (((w + 1) * EB) // NW) // 8 * 8)
    return lo, hi


NPH = NP // 2


def _sc_prep_body(row_hbm, col_hbm, degp_hbm, colm_hbm,
                  deg_v, rowv, colv, colmv, tmp_v, out_v, hist, sbuf):
    cc = lax.axis_index("c")
    ss = lax.axis_index("s")
    w = _wid()

    lo, hi = _blk_range(w)
    trash = N + lax.iota(jnp.int32, 16)
    lane = lax.iota(jnp.int32, 16)
    ones = jnp.ones((16,), jnp.float32)

    for p in range(2):
        def hz_body(i, _):
            hist[pl.ds(i * 16, 16)] = jnp.zeros((16,), jnp.float32)
            return ()
        lax.fori_loop(0, NS * NPH // 16, hz_body, ())

        def blk_body(b, _):
            pltpu.sync_copy(row_hbm.at[b], rowv)
            pltpu.sync_copy(col_hbm.at[b], colv)
            for j in range(8):
                rv = rowv[pl.ds(j * 16, 16)]
                cv = colv[pl.ds(j * 16, 16)]
                if p == 0:
                    colmv[pl.ds(j * 16, 16)] = jnp.where(rv == cv, trash, cv)
                local = rv - (p * NPH)
                m = (rv != cv) & (local >= 0) & (local < NPH)
                local = jnp.where(m, local, 0) + lane * NPH
                plsc.addupdate_scatter(hist, [local], ones, mask=m)
            if p == 0:
                pltpu.sync_copy(colmv, colm_hbm.at[b])
            return ()
        lax.fori_loop(lo, hi, blk_body, ())

        def red16_body(v, _):
            acc = hist[pl.ds(v * 16, 16)]
            for r in range(1, NS):
                acc = acc + hist[pl.ds(r * NPH + v * 16, 16)]
            deg_v[pl.ds(p * NPH + v * 16, 16)] = acc
            return ()
        lax.fori_loop(0, NPH // 16, red16_body, ())

    pltpu.sync_copy(deg_v, sbuf.at[ss])
    plsc.subcore_barrier()
    base = pl.multiple_of(ss * RPT, 8)
    for p in range(NS):
        pltpu.sync_copy(sbuf.at[p, pl.ds(base, RPT)], tmp_v.at[p])

    def red_body(v, _):
        acc = tmp_v[0, pl.ds(v * 16, 16)]
        for p in range(1, NS):
            acc = acc + tmp_v[p, pl.ds(v * 16, 16)]
        out_v[pl.ds(v * 16, 16)] = acc
        return ()
    lax.fori_loop(0, RPT // 16, red_body, ())
    pltpu.sync_copy(out_v, degp_hbm.at[cc, pl.ds(base, RPT)])


def _sc_prep(row2d, col2d):
    return pl.kernel(
        _sc_prep_body,
        out_type=[jax.ShapeDtypeStruct((NC, NP), jnp.float32),
                  jax.ShapeDtypeStruct((EBP, 128), jnp.int32)],
        mesh=plsc.VectorSubcoreMesh(**_MESH),
        scratch_types=[
            pltpu.VMEM((NP,), jnp.float32),
            pltpu.VMEM((128,), jnp.int32),
            pltpu.VMEM((128,), jnp.int32),
            pltpu.VMEM((128,), jnp.int32),
            pltpu.VMEM((NS, RPT), jnp.float32),
            pltpu.VMEM((RPT,), jnp.float32),
            pltpu.VMEM((NS * NPH,), jnp.float32),
            pltpu.VMEM_SHARED((NS, NP), jnp.float32),
        ],
        compiler_params=pltpu.CompilerParams(needs_layout_passes=False),
    )(row2d, col2d)


def _sc_prop_body(g_hbm, row_hbm, colm_hbm, s_hbm,
                  acc, zbuf, rowvb, colvb, rows, gsem, ssem):
    cc = lax.axis_index("c")
    ss = lax.axis_index("s")
    w = _wid()

    def zb_body(i, _):
        for j in range(8):
            zbuf[i, pl.ds(j * 16, 16)] = jnp.zeros((16,), jnp.float32)
        return ()
    lax.fori_loop(0, 64, zb_body, ())
    for t in range(RPT // 64):
        pltpu.sync_copy(
            zbuf, acc.at[pl.ds(pl.multiple_of(ss * RPT + t * 64, 8), 64)])
    plsc.subcore_barrier()

    lo, hi = _blk_range(w)
    n = hi - lo

    lo8 = pl.multiple_of(lo, 8)
    pltpu.sync_copy(row_hbm.at[pl.ds(lo8, 16)], rowvb.at[0])
    pltpu.sync_copy(colm_hbm.at[pl.ds(lo8, 16)], colvb.at[0])
    pltpu.async_copy(g_hbm.at[rowvb.at[0, 0]], rows.at[0], gsem.at[0])

    def body(k, _):
        kb = lax.rem(k, 2)
        r = lax.rem(k, 16)
        cp = lax.rem(k // 16, 2)

        @pl.when(k >= 1)
        def _():
            pb = lax.rem(k - 1, 2)
            pr = lax.rem(k - 1, 16)
            pcp = lax.rem((k - 1) // 16, 2)
            pltpu.make_async_copy(
                rows.at[pb], acc.at[colvb.at[pcp, pr]], ssem.at[pb]).wait()

        @pl.when(r == 0)
        def _():
            ncp = lax.rem(k // 16 + 1, 2)
            off = pl.multiple_of(lo + k + 16, 8)
            pltpu.sync_copy(row_hbm.at[pl.ds(off, 16)], rowvb.at[ncp])
            pltpu.sync_copy(colm_hbm.at[pl.ds(off, 16)], colvb.at[ncp])

        pltpu.make_async_copy(
            g_hbm.at[rowvb.at[cp, r]], rows.at[kb], gsem.at[kb]).wait()
        pltpu.async_copy(rows.at[kb], acc.at[colvb.at[cp, r]],
                         ssem.at[kb], add=True)

        @pl.when(k + 1 < n)
        def _():
            nb = lax.rem(k + 1, 2)
            nr = lax.rem(k + 1, 16)
            nc_ = lax.rem((k + 1) // 16, 2)
            pltpu.async_copy(g_hbm.at[rowvb.at[nc_, nr]], rows.at[nb],
                             gsem.at[nb])
        return ()
    lax.fori_loop(0, n, body, ())

    lb = lax.rem(n - 1, 2)
    lr = lax.rem(n - 1, 16)
    lcp = lax.rem((n - 1) // 16, 2)
    pltpu.make_async_copy(
        rows.at[lb], acc.at[colvb.at[lcp, lr]], ssem.at[lb]).wait()

    plsc.subcore_barrier()
    dof = pl.multiple_of(ss * RPT, 8)
    pltpu.sync_copy(acc.at[pl.ds(dof, RPT)], s_hbm.at[cc, pl.ds(dof, RPT)])


def _sc_prop(g, row2d, colm2d):
    return pl.kernel(
        _sc_prop_body,
        out_type=jax.ShapeDtypeStruct((NC, NP, F), jnp.float32),
        mesh=plsc.VectorSubcoreMesh(**_MESH),
        scratch_types=[
            pltpu.VMEM_SHARED((NP, F), jnp.float32),
            pltpu.VMEM((64, F), jnp.float32),
            pltpu.VMEM((2, 16, 128), jnp.int32),
            pltpu.VMEM((2, 16, 128), jnp.int32),
            pltpu.VMEM((2, 128, F), jnp.float32),
            pltpu.SemaphoreType.DMA((2,)),
            pltpu.SemaphoreType.DMA((2,)),
        ],
    )(g, row2d, colm2d)


BR = 512


def _tc_prep_body(degp_ref, xp_ref, dinv_ref, g0_ref):
    deg = degp_ref[0, :] + degp_ref[1, :]
    dv = jnp.where(deg > 0, lax.rsqrt(jnp.maximum(deg, 1.0)), 0.0)
    dinv_ref[:, 0] = dv
    g0_ref[...] = xp_ref[...] * dv[:, None]


def _tc_prep(degp, xp):
    return pl.pallas_call(
        _tc_prep_body,
        grid=(NP // BR,),
        in_specs=[pl.BlockSpec((NC, BR), lambda i: (0, i)),
                  pl.BlockSpec((BR, F), lambda i: (i, 0))],
        out_specs=[pl.BlockSpec((BR, 1), lambda i: (i, 0)),
                   pl.BlockSpec((BR, F), lambda i: (i, 0))],
        out_shape=[jax.ShapeDtypeStruct((NP, 1), jnp.float32),
                   jax.ShapeDtypeStruct((NP, F), jnp.float32)],
    )(degp, xp)


def _tc_combine_body(s_ref, dinv_ref, t1_ref, g1_ref):
    s = s_ref[0] + s_ref[1]
    dv = dinv_ref[...]
    t = (-dv) * s
    t1_ref[...] = t
    g1_ref[...] = dv * t


def _tc_combine(s, dinv):
    return pl.pallas_call(
        _tc_combine_body,
        grid=(NP // BR,),
        in_specs=[pl.BlockSpec((NC, BR, F), lambda i: (0, i, 0)),
                  pl.BlockSpec((BR, 1), lambda i: (i, 0))],
        out_specs=[pl.BlockSpec((BR, F), lambda i: (i, 0)),
                   pl.BlockSpec((BR, F), lambda i: (i, 0))],
        out_shape=[jax.ShapeDtypeStruct((NP, F), jnp.float32),
                   jax.ShapeDtypeStruct((NP, F), jnp.float32)],
    )(s, dinv)


def _tc_layer_body(h_ref, t1_ref, s2_ref, dinv_ref, w0_ref, w1_ref, w2_ref,
                   b_ref, hn_ref, gn_ref, *, relu):
    h = h_ref[...]
    dv = dinv_ref[...]
    t2 = (-2.0 * dv) * (s2_ref[0] + s2_ref[1]) - h
    z = jnp.dot(h, w0_ref[...], preferred_element_type=jnp.float32)
    z += jnp.dot(t1_ref[...], w1_ref[...], preferred_element_type=jnp.float32)
    z += jnp.dot(t2, w2_ref[...], preferred_element_type=jnp.float32)
    z += b_ref[...]
    if relu:
        z = jnp.clip(z, 0.0, 6.0)
    hn_ref[...] = z
    if gn_ref is not None:
        gn_ref[...] = dv * z


def _tc_layer(h, t1, s2, dinv, w, b, relu, want_g):
    nouts = 2 if want_g else 1
    body = functools.partial(_tc_layer_body, relu=relu)
    if not want_g:
        def body(h_ref, t1_ref, s2_ref, dinv_ref, w0, w1, w2, b_ref, hn_ref):
            _tc_layer_body(h_ref, t1_ref, s2_ref, dinv_ref, w0, w1, w2,
                           b_ref, hn_ref, None, relu=relu)
    outs = pl.pallas_call(
        body,
        grid=(NP // BR,),
        in_specs=[pl.BlockSpec((BR, F), lambda i: (i, 0)),
                  pl.BlockSpec((BR, F), lambda i: (i, 0)),
                  pl.BlockSpec((NC, BR, F), lambda i: (0, i, 0)),
                  pl.BlockSpec((BR, 1), lambda i: (i, 0)),
                  pl.BlockSpec((F, F), lambda i: (0, 0)),
                  pl.BlockSpec((F, F), lambda i: (0, 0)),
                  pl.BlockSpec((F, F), lambda i: (0, 0)),
                  pl.BlockSpec((1, F), lambda i: (0, 0))],
        out_specs=[pl.BlockSpec((BR, F), lambda i: (i, 0))] * nouts,
        out_shape=[jax.ShapeDtypeStruct((NP, F), jnp.float32)] * nouts,
    )(h, t1, s2, dinv, w[0], w[1], w[2], b.reshape(1, F))
    return outs if want_g else (outs[0] if isinstance(outs, (list, tuple)) else outs)


def kernel(x, edge_index, W1, b1, W2, b2, W3, b3, W4, b4):
    row2d = edge_index[0].astype(jnp.int32).reshape(EB, 128)
    col2d = edge_index[1].astype(jnp.int32).reshape(EB, 128)
    row2d = jnp.pad(row2d, ((0, EBP - EB), (0, 0)))
    col2d = jnp.pad(col2d, ((0, EBP - EB), (0, 0)))
    xp = jnp.zeros((NP, F), x.dtype).at[:N].set(x)

    degp, colm2d = _sc_prep(row2d, col2d)
    dinv, g = _tc_prep(degp, xp)

    h = xp
    layers = [(W1, b1, True), (W2, b2, True), (W3, b3, True)]
    for (w, b, relu) in layers:
        s1 = _sc_prop(g, row2d, colm2d)
        t1, g1 = _tc_combine(s1, dinv)
        s2 = _sc_prop(g1, row2d, colm2d)
        h, g = _tc_layer(h, t1, s2, dinv, w, b, relu, True)

    w4p = jnp.zeros((3, F, F), jnp.float32).at[:, :, :W4.shape[2]].set(W4)
    b4p = jnp.zeros((F,), jnp.float32).at[:W4.shape[2]].set(b4)
    s1 = _sc_prop(g, row2d, colm2d)
    t1, g1 = _tc_combine(s1, dinv)
    s2 = _sc_prop(g1, row2d, colm2d)
    out = _tc_layer(h, t1, s2, dinv, w4p, b4p, False, False)
    return (out[:N, :W4.shape[2]], edge_index)


# speedup vs baseline: 12.4344x; 1.1288x over previous
import functools

import jax
import jax.numpy as jnp
from jax import lax
from jax.experimental import pallas as pl
from jax.experimental.pallas import tpu as pltpu
from jax.experimental.pallas import tpu_sc as plsc

N = 10000
E = 320000
F = 128
NP = 10240
EB = E // 128
EBP = 2560
EB2 = E // 64
EBP2 = 2 * EBP
NC, NS = 2, 16
NW = NC * NS
RPT = NP // NS

_MESH = dict(core_axis_name="c", subcore_axis_name="s", num_cores=NC,
             num_subcores=NS)


def _wid():
    return lax.axis_index("s") * NC + lax.axis_index("c")


def _blk_range(w):
    lo = ((w * EB) // NW) // 8 * 8
    hi = jnp.where(w == NW - 1, EB, (((w + 1) * EB) // NW) // 8 * 8)
    return lo, hi


def _blk_range2(w):
    lo = ((w * EB2) // NW) // 8 * 8
    hi = jnp.where(w == NW - 1, EB2, (((w + 1) * EB2) // NW) // 8 * 8)
    return lo, hi


def _sc_prep_body(row_hbm, col_hbm, degp_hbm, colm_hbm,
                  dacc, zbuf, ones_v, rowcb, colcb, rowmb, colmb, ssem):
    cc = lax.axis_index("c")
    ss = lax.axis_index("s")
    w = _wid()

    for j in range(8):
        ones_v[pl.ds(j * 16, 16)] = jnp.ones((16,), jnp.float32)

    def z_body(i, _):
        zbuf[pl.ds(i * 16, 16)] = jnp.zeros((16,), jnp.float32)
        return ()
    lax.fori_loop(0, RPT // 16, z_body, ())
    pltpu.sync_copy(zbuf, dacc.at[pl.ds(pl.multiple_of(ss * RPT, 8), RPT)])
    plsc.subcore_barrier()

    lo, hi = _blk_range(w)
    n = hi - lo
    nch = (n + 7) // 8
    trash = N + lax.iota(jnp.int32, 16)

    def chunk_body(c, _):
        off = pl.multiple_of(lo + c * 8, 8)
        pltpu.sync_copy(row_hbm.at[pl.ds(off, 8)], rowcb)
        pltpu.sync_copy(col_hbm.at[pl.ds(off, 8)], colcb)
        for q in range(8):
            for j in range(8):
                rv = rowcb[q, pl.ds(j * 16, 16)]
                cv = colcb[q, pl.ds(j * 16, 16)]
                eq = rv == cv
                rowmb[q, pl.ds(j * 16, 16)] = jnp.where(eq, trash, rv)
                colmb[q, pl.ds(j * 16, 16)] = jnp.where(eq, trash, cv)
            pltpu.async_copy(ones_v, dacc.at[rowmb.at[q]], ssem, add=True)
        pltpu.sync_copy(colmb, colm_hbm.at[pl.ds(off, 8)])
        for q in range(8):
            pltpu.make_async_copy(
                ones_v, dacc.at[rowmb.at[q]], ssem).wait()
        return ()
    lax.fori_loop(0, nch, chunk_body, ())

    plsc.subcore_barrier()
    dof = pl.multiple_of(ss * RPT, 8)
    pltpu.sync_copy(dacc.at[pl.ds(dof, RPT)], degp_hbm.at[cc, pl.ds(dof, RPT)])


def _sc_prep(row2d, col2d):
    return pl.kernel(
        _sc_prep_body,
        out_type=[jax.ShapeDtypeStruct((NC, NP), jnp.float32),
                  jax.ShapeDtypeStruct((EBP, 128), jnp.int32)],
        mesh=plsc.VectorSubcoreMesh(**_MESH),
        scratch_types=[
            pltpu.VMEM_SHARED((NP,), jnp.float32),
            pltpu.VMEM((RPT,), jnp.float32),
            pltpu.VMEM((128,), jnp.float32),
            pltpu.VMEM((8, 128), jnp.int32),
            pltpu.VMEM((8, 128), jnp.int32),
            pltpu.VMEM((8, 128), jnp.int32),
            pltpu.VMEM((8, 128), jnp.int32),
            pltpu.SemaphoreType.DMA,
        ],
        compiler_params=pltpu.CompilerParams(needs_layout_passes=False),
    )(row2d, col2d)


def _sc_prop_body(g_hbm, row_hbm, colm_hbm, s_hbm,
                  acc, rowvb, colvb, rows, gsem, ssem):
    cc = lax.axis_index("c")
    ss = lax.axis_index("s")
    w = _wid()

    lo, hi = _blk_range2(w)
    n = hi - lo

    def _idx(k):
        return lax.rem(k, 4), lax.rem(k, 8), lax.rem(k // 8, 4)

    def _gather(k):
        kb, r, cp = _idx(k)
        return g_hbm.at[rowvb.at[cp, r]], rows.at[kb], gsem.at[kb]

    def _scatter(k):
        kb, r, cp = _idx(k)
        return rows.at[kb], acc.at[colvb.at[cp, r]], ssem.at[kb]

    def _load_chunk(base, slot):
        off = pl.multiple_of(base, 8)
        pltpu.sync_copy(row_hbm.at[pl.ds(off, 8)], rowvb.at[slot])
        pltpu.sync_copy(colm_hbm.at[pl.ds(off, 8)], colvb.at[slot])

    def zb_body(i, _):
        for j in range(8):
            rows[0, i, pl.ds(j * 16, 16)] = jnp.zeros((16,), jnp.float32)
        return ()
    lax.fori_loop(0, 16, zb_body, ())

    def zc_body(t, _):
        pltpu.sync_copy(
            rows.at[0, pl.ds(0, 16)],
            acc.at[pl.ds(pl.multiple_of(ss * RPT + t * 16, 8), 16)])
        return ()
    lax.fori_loop(0, RPT // 16, zc_body, ())

    lo8 = pl.multiple_of(lo, 8)
    _load_chunk(lo8, 0)
    _load_chunk(lo8 + 8, 1)
    for k0 in range(2):
        pltpu.async_copy(*_gather(k0))
    plsc.subcore_barrier()

    def body(k, _):
        @pl.when(k >= 2)
        def _():
            s, d, m = _scatter(k - 2)
            pltpu.make_async_copy(s, d, m).wait()

        @pl.when(lax.rem(k, 8) == 0)
        def _():
            j = k // 8
            _load_chunk(lo8 + (j + 2) * 8, lax.rem(j + 2, 4))

        s, d, m = _gather(k)
        pltpu.make_async_copy(s, d, m).wait()
        pltpu.async_copy(*_scatter(k), add=True)

        @pl.when(k + 2 < n)
        def _():
            pltpu.async_copy(*_gather(k + 2))
        return ()
    lax.fori_loop(0, n, body, ())

    for tail in range(2):
        k = n - 2 + tail
        s, d, m = _scatter(k)
        pltpu.make_async_copy(s, d, m).wait()

    plsc.subcore_barrier()
    def dump_body(t, _):
        dof = pl.multiple_of(ss * RPT + t * 64, 8)
        pltpu.sync_copy(acc.at[pl.ds(dof, 64)],
                        s_hbm.at[cc, pl.ds(dof, 64)])
        return ()
    lax.fori_loop(0, RPT // 64, dump_body, ())


def _sc_prop(g, row2d, colm2d):
    return pl.kernel(
        _sc_prop_body,
        out_type=jax.ShapeDtypeStruct((NC, NP, F), jnp.float32),
        mesh=plsc.VectorSubcoreMesh(**_MESH),
        scratch_types=[
            pltpu.VMEM_SHARED((NP, F), jnp.float32),
            pltpu.VMEM((4, 8, 64), jnp.int32),
            pltpu.VMEM((4, 8, 64), jnp.int32),
            pltpu.VMEM((4, 64, F), jnp.float32),
            pltpu.SemaphoreType.DMA((4,)),
            pltpu.SemaphoreType.DMA((4,)),
        ],
    )(g, row2d, colm2d)


BR = 512


def _tc_prep_body(degp_ref, xp_ref, dinv_ref, g0_ref):
    deg = degp_ref[0, :] + degp_ref[1, :]
    dv = jnp.where(deg > 0, lax.rsqrt(jnp.maximum(deg, 1.0)), 0.0)
    dinv_ref[:, 0] = dv
    g0_ref[...] = xp_ref[...] * dv[:, None]


def _tc_prep(degp, xp):
    return pl.pallas_call(
        _tc_prep_body,
        grid=(NP // BR,),
        in_specs=[pl.BlockSpec((NC, BR), lambda i: (0, i)),
                  pl.BlockSpec((BR, F), lambda i: (i, 0))],
        out_specs=[pl.BlockSpec((BR, 1), lambda i: (i, 0)),
                   pl.BlockSpec((BR, F), lambda i: (i, 0))],
        out_shape=[jax.ShapeDtypeStruct((NP, 1), jnp.float32),
                   jax.ShapeDtypeStruct((NP, F), jnp.float32)],
    )(degp, xp)


def _tc_combine_body(s_ref, dinv_ref, t1_ref, g1_ref):
    s = s_ref[0] + s_ref[1]
    dv = dinv_ref[...]
    t = (-dv) * s
    t1_ref[...] = t
    g1_ref[...] = dv * t


def _tc_combine(s, dinv):
    return pl.pallas_call(
        _tc_combine_body,
        grid=(NP // BR,),
        in_specs=[pl.BlockSpec((NC, BR, F), lambda i: (0, i, 0)),
                  pl.BlockSpec((BR, 1), lambda i: (i, 0))],
        out_specs=[pl.BlockSpec((BR, F), lambda i: (i, 0)),
                   pl.BlockSpec((BR, F), lambda i: (i, 0))],
        out_shape=[jax.ShapeDtypeStruct((NP, F), jnp.float32),
                   jax.ShapeDtypeStruct((NP, F), jnp.float32)],
    )(s, dinv)


def _tc_layer_body(h_ref, t1_ref, s2_ref, dinv_ref, w0_ref, w1_ref, w2_ref,
                   b_ref, hn_ref, gn_ref, *, relu):
    h = h_ref[...]
    dv = dinv_ref[...]
    t2 = (-2.0 * dv) * (s2_ref[0] + s2_ref[1]) - h
    z = jnp.dot(h, w0_ref[...], preferred_element_type=jnp.float32)
    z += jnp.dot(t1_ref[...], w1_ref[...], preferred_element_type=jnp.float32)
    z += jnp.dot(t2, w2_ref[...], preferred_element_type=jnp.float32)
    z += b_ref[...]
    if relu:
        z = jnp.clip(z, 0.0, 6.0)
    hn_ref[...] = z
    if gn_ref is not None:
        gn_ref[...] = dv * z


def _tc_layer(h, t1, s2, dinv, w, b, relu, want_g):
    nouts = 2 if want_g else 1
    body = functools.partial(_tc_layer_body, relu=relu)
    if not want_g:
        def body(h_ref, t1_ref, s2_ref, dinv_ref, w0, w1, w2, b_ref, hn_ref):
            _tc_layer_body(h_ref, t1_ref, s2_ref, dinv_ref, w0, w1, w2,
                           b_ref, hn_ref, None, relu=relu)
    outs = pl.pallas_call(
        body,
        grid=(NP // BR,),
        in_specs=[pl.BlockSpec((BR, F), lambda i: (i, 0)),
                  pl.BlockSpec((BR, F), lambda i: (i, 0)),
                  pl.BlockSpec((NC, BR, F), lambda i: (0, i, 0)),
                  pl.BlockSpec((BR, 1), lambda i: (i, 0)),
                  pl.BlockSpec((F, F), lambda i: (0, 0)),
                  pl.BlockSpec((F, F), lambda i: (0, 0)),
                  pl.BlockSpec((F, F), lambda i: (0, 0)),
                  pl.BlockSpec((1, F), lambda i: (0, 0))],
        out_specs=[pl.BlockSpec((BR, F), lambda i: (i, 0))] * nouts,
        out_shape=[jax.ShapeDtypeStruct((NP, F), jnp.float32)] * nouts,
    )(h, t1, s2, dinv, w[0], w[1], w[2], b.reshape(1, F))
    return outs if want_g else (outs[0] if isinstance(outs, (list, tuple)) else outs)


def kernel(x, edge_index, W1, b1, W2, b2, W3, b3, W4, b4):
    row2d = edge_index[0].astype(jnp.int32).reshape(EB, 128)
    col2d = edge_index[1].astype(jnp.int32).reshape(EB, 128)
    row2d = jnp.pad(row2d, ((0, EBP - EB), (0, 0)))
    col2d = jnp.pad(col2d, ((0, EBP - EB), (0, 0)))
    xp = jnp.zeros((NP, F), x.dtype).at[:N].set(x)

    degp, colm2d = _sc_prep(row2d, col2d)
    dinv, g = _tc_prep(degp, xp)
    row64 = row2d.reshape(EBP2, 64)
    colm64 = colm2d.reshape(EBP2, 64)

    h = xp
    layers = [(W1, b1, True), (W2, b2, True), (W3, b3, True)]
    for (w, b, relu) in layers:
        s1 = _sc_prop(g, row64, colm64)
        t1, g1 = _tc_combine(s1, dinv)
        s2 = _sc_prop(g1, row64, colm64)
        h, g = _tc_layer(h, t1, s2, dinv, w, b, relu, True)

    w4p = jnp.zeros((3, F, F), jnp.float32).at[:, :, :W4.shape[2]].set(W4)
    b4p = jnp.zeros((F,), jnp.float32).at[:W4.shape[2]].set(b4)
    s1 = _sc_prop(g, row64, colm64)
    t1, g1 = _tc_combine(s1, dinv)
    s2 = _sc_prop(g1, row64, colm64)
    out = _tc_layer(h, t1, s2, dinv, w4p, b4p, False, False)
    return (out[:N, :W4.shape[2]], edge_index)


# speedup vs baseline: 12.5047x; 1.0056x over previous
import functools

import jax
import jax.numpy as jnp
from jax import lax
from jax.experimental import pallas as pl
from jax.experimental.pallas import tpu as pltpu
from jax.experimental.pallas import tpu_sc as plsc

N = 10000
E = 320000
F = 128
NP = 10240
EB = E // 128
EBP = 2560
EB2 = E // 64
EBP2 = 2 * EBP
NC, NS = 2, 16
NW = NC * NS
RPT = NP // NS

_MESH = dict(core_axis_name="c", subcore_axis_name="s", num_cores=NC,
             num_subcores=NS)


def _wid():
    return lax.axis_index("s") * NC + lax.axis_index("c")


def _blk_range(w):
    lo = ((w * EB) // NW) // 8 * 8
    hi = jnp.where(w == NW - 1, EB, (((w + 1) * EB) // NW) // 8 * 8)
    return lo, hi


def _blk_range2(w):
    lo = ((w * EB2) // NW) // 8 * 8
    hi = jnp.where(w == NW - 1, EB2, (((w + 1) * EB2) // NW) // 8 * 8)
    return lo, hi


def _sc_prep_body(row_hbm, col_hbm, degp_hbm, colm_hbm,
                  dacc, zbuf, ones_v, rowcb, colcb, rowmb, colmb, ssem):
    cc = lax.axis_index("c")
    ss = lax.axis_index("s")
    w = _wid()

    for j in range(8):
        ones_v[pl.ds(j * 16, 16)] = jnp.ones((16,), jnp.float32)

    def z_body(i, _):
        zbuf[pl.ds(i * 16, 16)] = jnp.zeros((16,), jnp.float32)
        return ()
    lax.fori_loop(0, RPT // 16, z_body, ())
    pltpu.sync_copy(zbuf, dacc.at[pl.ds(pl.multiple_of(ss * RPT, 8), RPT)])
    plsc.subcore_barrier()

    lo, hi = _blk_range(w)
    n = hi - lo
    nch = (n + 7) // 8
    trash = N + lax.iota(jnp.int32, 16)

    def chunk_body(c, _):
        off = pl.multiple_of(lo + c * 8, 8)
        pltpu.sync_copy(row_hbm.at[pl.ds(off, 8)], rowcb)
        pltpu.sync_copy(col_hbm.at[pl.ds(off, 8)], colcb)
        for q in range(8):
            for j in range(8):
                rv = rowcb[q, pl.ds(j * 16, 16)]
                cv = colcb[q, pl.ds(j * 16, 16)]
                eq = rv == cv
                rowmb[q, pl.ds(j * 16, 16)] = jnp.where(eq, trash, rv)
                colmb[q, pl.ds(j * 16, 16)] = jnp.where(eq, trash, cv)
            pltpu.async_copy(ones_v, dacc.at[rowmb.at[q]], ssem, add=True)
        pltpu.sync_copy(colmb, colm_hbm.at[pl.ds(off, 8)])
        for q in range(8):
            pltpu.make_async_copy(
                ones_v, dacc.at[rowmb.at[q]], ssem).wait()
        return ()
    lax.fori_loop(0, nch, chunk_body, ())

    plsc.subcore_barrier()
    dof = pl.multiple_of(ss * RPT, 8)
    pltpu.sync_copy(dacc.at[pl.ds(dof, RPT)], degp_hbm.at[cc, pl.ds(dof, RPT)])


def _sc_prep(row2d, col2d):
    return pl.kernel(
        _sc_prep_body,
        out_type=[jax.ShapeDtypeStruct((NC, NP), jnp.float32),
                  jax.ShapeDtypeStruct((EBP, 128), jnp.int32)],
        mesh=plsc.VectorSubcoreMesh(**_MESH),
        scratch_types=[
            pltpu.VMEM_SHARED((NP,), jnp.float32),
            pltpu.VMEM((RPT,), jnp.float32),
            pltpu.VMEM((128,), jnp.float32),
            pltpu.VMEM((8, 128), jnp.int32),
            pltpu.VMEM((8, 128), jnp.int32),
            pltpu.VMEM((8, 128), jnp.int32),
            pltpu.VMEM((8, 128), jnp.int32),
            pltpu.SemaphoreType.DMA,
        ],
        compiler_params=pltpu.CompilerParams(needs_layout_passes=False),
    )(row2d, col2d)


def _sc_prop_body(g_hbm, row_hbm, colm_hbm, s_hbm,
                  acc, rowvb, colvb, rows, gsem, ssem):
    cc = lax.axis_index("c")
    ss = lax.axis_index("s")
    w = _wid()

    lo, hi = _blk_range2(w)
    n = hi - lo

    def _idx(k):
        return lax.rem(k, 4), lax.rem(k, 8), lax.rem(k // 8, 4)

    def _gather(k):
        kb, r, cp = _idx(k)
        return g_hbm.at[rowvb.at[cp, r]], rows.at[kb], gsem.at[kb]

    def _scatter(k):
        kb, r, cp = _idx(k)
        return rows.at[kb], acc.at[colvb.at[cp, r]], ssem.at[kb]

    def _load_chunk(base, slot):
        off = pl.multiple_of(base, 8)
        pltpu.sync_copy(row_hbm.at[pl.ds(off, 8)], rowvb.at[slot])
        pltpu.sync_copy(colm_hbm.at[pl.ds(off, 8)], colvb.at[slot])

    def zb_body(i, _):
        for j in range(8):
            rows[0, i, pl.ds(j * 16, 16)] = jnp.zeros((16,), jnp.float32)
        return ()
    lax.fori_loop(0, 64, zb_body, ())

    def zc_body(t, _):
        pltpu.sync_copy(
            rows.at[0],
            acc.at[pl.ds(pl.multiple_of(ss * RPT + t * 64, 8), 64)])
        return ()
    lax.fori_loop(0, RPT // 64, zc_body, ())

    lo8 = pl.multiple_of(lo, 8)
    _load_chunk(lo8, 0)
    _load_chunk(lo8 + 8, 1)
    for k0 in range(2):
        pltpu.async_copy(*_gather(k0))
    plsc.subcore_barrier()

    def body(k, _):
        @pl.when(k >= 2)
        def _():
            s, d, m = _scatter(k - 2)
            pltpu.make_async_copy(s, d, m).wait()

        @pl.when(lax.rem(k, 8) == 0)
        def _():
            j = k // 8
            _load_chunk(lo8 + (j + 2) * 8, lax.rem(j + 2, 4))

        s, d, m = _gather(k)
        pltpu.make_async_copy(s, d, m).wait()
        pltpu.async_copy(*_scatter(k), add=True)

        @pl.when(k + 2 < n)
        def _():
            pltpu.async_copy(*_gather(k + 2))
        return ()
    lax.fori_loop(0, n, body, ())

    for tail in range(2):
        k = n - 2 + tail
        s, d, m = _scatter(k)
        pltpu.make_async_copy(s, d, m).wait()

    plsc.subcore_barrier()
    dof = pl.multiple_of(ss * RPT, 8)
    pltpu.sync_copy(acc.at[pl.ds(dof, RPT)], s_hbm.at[cc, pl.ds(dof, RPT)])


def _sc_prop(g, row2d, colm2d):
    return pl.kernel(
        _sc_prop_body,
        out_type=jax.ShapeDtypeStruct((NC, NP, F), jnp.float32),
        mesh=plsc.VectorSubcoreMesh(**_MESH),
        scratch_types=[
            pltpu.VMEM_SHARED((NP, F), jnp.float32),
            pltpu.VMEM((4, 8, 64), jnp.int32),
            pltpu.VMEM((4, 8, 64), jnp.int32),
            pltpu.VMEM((4, 64, F), jnp.float32),
            pltpu.SemaphoreType.DMA((4,)),
            pltpu.SemaphoreType.DMA((4,)),
        ],
    )(g, row2d, colm2d)


BR = 512


def _tc_prep_body(degp_ref, xp_ref, dinv_ref, g0_ref):
    deg = degp_ref[0, :] + degp_ref[1, :]
    dv = jnp.where(deg > 0, lax.rsqrt(jnp.maximum(deg, 1.0)), 0.0)
    dinv_ref[:, 0] = dv
    g0_ref[...] = xp_ref[...] * dv[:, None]


def _tc_prep(degp, xp):
    return pl.pallas_call(
        _tc_prep_body,
        grid=(NP // BR,),
        in_specs=[pl.BlockSpec((NC, BR), lambda i: (0, i)),
                  pl.BlockSpec((BR, F), lambda i: (i, 0))],
        out_specs=[pl.BlockSpec((BR, 1), lambda i: (i, 0)),
                   pl.BlockSpec((BR, F), lambda i: (i, 0))],
        out_shape=[jax.ShapeDtypeStruct((NP, 1), jnp.float32),
                   jax.ShapeDtypeStruct((NP, F), jnp.float32)],
    )(degp, xp)


def _tc_combine_body(s_ref, dinv_ref, t1_ref, g1_ref):
    s = s_ref[0] + s_ref[1]
    dv = dinv_ref[...]
    t = (-dv) * s
    t1_ref[...] = t
    g1_ref[...] = dv * t


def _tc_combine(s, dinv):
    return pl.pallas_call(
        _tc_combine_body,
        grid=(NP // BR,),
        in_specs=[pl.BlockSpec((NC, BR, F), lambda i: (0, i, 0)),
                  pl.BlockSpec((BR, 1), lambda i: (i, 0))],
        out_specs=[pl.BlockSpec((BR, F), lambda i: (i, 0)),
                   pl.BlockSpec((BR, F), lambda i: (i, 0))],
        out_shape=[jax.ShapeDtypeStruct((NP, F), jnp.float32),
                   jax.ShapeDtypeStruct((NP, F), jnp.float32)],
    )(s, dinv)


def _tc_layer_body(h_ref, t1_ref, s2_ref, dinv_ref, w0_ref, w1_ref, w2_ref,
                   b_ref, hn_ref, gn_ref, *, relu):
    h = h_ref[...]
    dv = dinv_ref[...]
    t2 = (-2.0 * dv) * (s2_ref[0] + s2_ref[1]) - h
    z = jnp.dot(h, w0_ref[...], preferred_element_type=jnp.float32)
    z += jnp.dot(t1_ref[...], w1_ref[...], preferred_element_type=jnp.float32)
    z += jnp.dot(t2, w2_ref[...], preferred_element_type=jnp.float32)
    z += b_ref[...]
    if relu:
        z = jnp.clip(z, 0.0, 6.0)
    hn_ref[...] = z
    if gn_ref is not None:
        gn_ref[...] = dv * z


def _tc_layer(h, t1, s2, dinv, w, b, relu, want_g):
    nouts = 2 if want_g else 1
    body = functools.partial(_tc_layer_body, relu=relu)
    if not want_g:
        def body(h_ref, t1_ref, s2_ref, dinv_ref, w0, w1, w2, b_ref, hn_ref):
            _tc_layer_body(h_ref, t1_ref, s2_ref, dinv_ref, w0, w1, w2,
                           b_ref, hn_ref, None, relu=relu)
    outs = pl.pallas_call(
        body,
        grid=(NP // BR,),
        in_specs=[pl.BlockSpec((BR, F), lambda i: (i, 0)),
                  pl.BlockSpec((BR, F), lambda i: (i, 0)),
                  pl.BlockSpec((NC, BR, F), lambda i: (0, i, 0)),
                  pl.BlockSpec((BR, 1), lambda i: (i, 0)),
                  pl.BlockSpec((F, F), lambda i: (0, 0)),
                  pl.BlockSpec((F, F), lambda i: (0, 0)),
                  pl.BlockSpec((F, F), lambda i: (0, 0)),
                  pl.BlockSpec((1, F), lambda i: (0, 0))],
        out_specs=[pl.BlockSpec((BR, F), lambda i: (i, 0))] * nouts,
        out_shape=[jax.ShapeDtypeStruct((NP, F), jnp.float32)] * nouts,
    )(h, t1, s2, dinv, w[0], w[1], w[2], b.reshape(1, F))
    return outs if want_g else (outs[0] if isinstance(outs, (list, tuple)) else outs)


def kernel(x, edge_index, W1, b1, W2, b2, W3, b3, W4, b4):
    row2d = edge_index[0].astype(jnp.int32).reshape(EB, 128)
    col2d = edge_index[1].astype(jnp.int32).reshape(EB, 128)
    row2d = jnp.pad(row2d, ((0, EBP - EB), (0, 0)))
    col2d = jnp.pad(col2d, ((0, EBP - EB), (0, 0)))
    xp = jnp.zeros((NP, F), x.dtype).at[:N].set(x)

    degp, colm2d = _sc_prep(row2d, col2d)
    dinv, g = _tc_prep(degp, xp)
    row64 = row2d.reshape(EBP2, 64)
    colm64 = colm2d.reshape(EBP2, 64)

    h = xp
    layers = [(W1, b1, True), (W2, b2, True), (W3, b3, True)]
    for (w, b, relu) in layers:
        s1 = _sc_prop(g, row64, colm64)
        t1, g1 = _tc_combine(s1, dinv)
        s2 = _sc_prop(g1, row64, colm64)
        h, g = _tc_layer(h, t1, s2, dinv, w, b, relu, True)

    w4p = jnp.zeros((3, F, F), jnp.float32).at[:, :, :W4.shape[2]].set(W4)
    b4p = jnp.zeros((F,), jnp.float32).at[:W4.shape[2]].set(b4)
    s1 = _sc_prop(g, row64, colm64)
    t1, g1 = _tc_combine(s1, dinv)
    s2 = _sc_prop(g1, row64, colm64)
    out = _tc_layer(h, t1, s2, dinv, w4p, b4p, False, False)
    return (out[:N, :W4.shape[2]], edge_index)


# speedup vs baseline: 13.7660x; 1.1009x over previous
import functools

import jax
import jax.numpy as jnp
from jax import lax
from jax.experimental import pallas as pl
from jax.experimental.pallas import tpu as pltpu
from jax.experimental.pallas import tpu_sc as plsc

N = 10000
E = 320000
F = 128
NP = 10240
EB = E // 128
EBP = 2560
EB2 = E // 64
EBP2 = 2 * EBP
NC, NS = 2, 16
NW = NC * NS
RPT = NP // NS

_MESH = dict(core_axis_name="c", subcore_axis_name="s", num_cores=NC,
             num_subcores=NS)


def _wid():
    return lax.axis_index("s") * NC + lax.axis_index("c")


def _blk_range(w):
    lo = ((w * EB) // NW) // 8 * 8
    hi = jnp.where(w == NW - 1, EB, (((w + 1) * EB) // NW) // 8 * 8)
    return lo, hi


def _blk_range2(w):
    lo = ((w * EB2) // NW) // 8 * 8
    hi = jnp.where(w == NW - 1, EB2, (((w + 1) * EB2) // NW) // 8 * 8)
    return lo, hi


def _sc_prep_body(row_hbm, col_hbm, degp_hbm, colm_hbm,
                  dacc, zbuf, ones_v, rowcb, colcb, rowmb, colmb, ssem):
    cc = lax.axis_index("c")
    ss = lax.axis_index("s")
    w = _wid()

    for j in range(8):
        ones_v[pl.ds(j * 16, 16)] = jnp.ones((16,), jnp.float32)

    def z_body(i, _):
        zbuf[pl.ds(i * 16, 16)] = jnp.zeros((16,), jnp.float32)
        return ()
    lax.fori_loop(0, RPT // 16, z_body, ())
    pltpu.sync_copy(zbuf, dacc.at[pl.ds(pl.multiple_of(ss * RPT, 8), RPT)])
    plsc.subcore_barrier()

    lo, hi = _blk_range(w)
    n = hi - lo
    nch = (n + 7) // 8
    trash = N + lax.iota(jnp.int32, 16)

    def chunk_body(c, _):
        off = pl.multiple_of(lo + c * 8, 8)
        pltpu.sync_copy(row_hbm.at[pl.ds(off, 8)], rowcb)
        pltpu.sync_copy(col_hbm.at[pl.ds(off, 8)], colcb)
        for q in range(8):
            for j in range(8):
                rv = rowcb[q, pl.ds(j * 16, 16)]
                cv = colcb[q, pl.ds(j * 16, 16)]
                eq = rv == cv
                rowmb[q, pl.ds(j * 16, 16)] = jnp.where(eq, trash, rv)
                colmb[q, pl.ds(j * 16, 16)] = jnp.where(eq, trash, cv)
            pltpu.async_copy(ones_v, dacc.at[rowmb.at[q]], ssem, add=True)
        pltpu.sync_copy(colmb, colm_hbm.at[pl.ds(off, 8)])
        for q in range(8):
            pltpu.make_async_copy(
                ones_v, dacc.at[rowmb.at[q]], ssem).wait()
        return ()
    lax.fori_loop(0, nch, chunk_body, ())

    plsc.subcore_barrier()
    dof = pl.multiple_of(ss * RPT, 8)
    pltpu.sync_copy(dacc.at[pl.ds(dof, RPT)], degp_hbm.at[cc, pl.ds(dof, RPT)])


def _sc_prep(row2d, col2d):
    return pl.kernel(
        _sc_prep_body,
        out_type=[jax.ShapeDtypeStruct((NC, NP), jnp.float32),
                  jax.ShapeDtypeStruct((EBP, 128), jnp.int32)],
        mesh=plsc.VectorSubcoreMesh(**_MESH),
        scratch_types=[
            pltpu.VMEM_SHARED((NP,), jnp.float32),
            pltpu.VMEM((RPT,), jnp.float32),
            pltpu.VMEM((128,), jnp.float32),
            pltpu.VMEM((8, 128), jnp.int32),
            pltpu.VMEM((8, 128), jnp.int32),
            pltpu.VMEM((8, 128), jnp.int32),
            pltpu.VMEM((8, 128), jnp.int32),
            pltpu.SemaphoreType.DMA,
        ],
        compiler_params=pltpu.CompilerParams(needs_layout_passes=False),
    )(row2d, col2d)


def _sc_prop_body(g_hbm, row_hbm, colm_hbm, s_hbm,
                  acc, rowvb, colvb, rows, gsem, ssem, csem):
    cc = lax.axis_index("c")
    ss = lax.axis_index("s")
    w = _wid()

    lo, hi = _blk_range2(w)
    n = hi - lo

    def _idx(k):
        return lax.rem(k, 4), lax.rem(k, 8), lax.rem(k // 8, 4)

    def _gather(k):
        kb, r, cp = _idx(k)
        return g_hbm.at[rowvb.at[cp, r]], rows.at[kb], gsem.at[kb]

    def _scatter(k):
        kb, r, cp = _idx(k)
        return rows.at[kb], acc.at[colvb.at[cp, r]], ssem.at[kb]

    def _load_chunk(base, slot):
        off = pl.multiple_of(base, 8)
        pltpu.async_copy(row_hbm.at[pl.ds(off, 8)], rowvb.at[slot],
                         csem.at[slot])
        pltpu.async_copy(colm_hbm.at[pl.ds(off, 8)], colvb.at[slot],
                         csem.at[slot])

    def _wait_chunk(base, slot):
        off = pl.multiple_of(base, 8)
        pltpu.make_async_copy(row_hbm.at[pl.ds(off, 8)], rowvb.at[slot],
                              csem.at[slot]).wait()
        pltpu.make_async_copy(colm_hbm.at[pl.ds(off, 8)], colvb.at[slot],
                              csem.at[slot]).wait()

    def zb_body(i, _):
        for j in range(8):
            rows[0, i, pl.ds(j * 16, 16)] = jnp.zeros((16,), jnp.float32)
        return ()
    lax.fori_loop(0, 64, zb_body, ())

    def zc_body(t, _):
        pltpu.sync_copy(
            rows.at[0],
            acc.at[pl.ds(pl.multiple_of(ss * RPT + t * 64, 8), 64)])
        return ()
    lax.fori_loop(0, RPT // 64, zc_body, ())

    lo8 = pl.multiple_of(lo, 8)
    _load_chunk(lo8, 0)
    _load_chunk(lo8 + 8, 1)
    _wait_chunk(lo8, 0)
    for k0 in range(2):
        pltpu.async_copy(*_gather(k0))
    plsc.subcore_barrier()

    def body(k, _):
        @pl.when(k >= 2)
        def _():
            s, d, m = _scatter(k - 2)
            pltpu.make_async_copy(s, d, m).wait()

        @pl.when(lax.rem(k, 8) == 0)
        def _():
            j = k // 8
            _wait_chunk(lo8 + (j + 1) * 8, lax.rem(j + 1, 4))
            _load_chunk(lo8 + (j + 2) * 8, lax.rem(j + 2, 4))

        s, d, m = _gather(k)
        pltpu.make_async_copy(s, d, m).wait()
        pltpu.async_copy(*_scatter(k), add=True)

        @pl.when(k + 2 < n)
        def _():
            pltpu.async_copy(*_gather(k + 2))
        return ()
    lax.fori_loop(0, n, body, ())

    for tail in range(2):
        k = n - 2 + tail
        s, d, m = _scatter(k)
        pltpu.make_async_copy(s, d, m).wait()
    jl = (n - 1) // 8 + 2
    _wait_chunk(lo8 + jl * 8, lax.rem(jl, 4))

    plsc.subcore_barrier()
    dof = pl.multiple_of(ss * RPT, 8)
    pltpu.sync_copy(acc.at[pl.ds(dof, RPT)], s_hbm.at[cc, pl.ds(dof, RPT)])


def _sc_prop(g, row2d, colm2d):
    return pl.kernel(
        _sc_prop_body,
        out_type=jax.ShapeDtypeStruct((NC, NP, F), jnp.float32),
        mesh=plsc.VectorSubcoreMesh(**_MESH),
        scratch_types=[
            pltpu.VMEM_SHARED((NP, F), jnp.float32),
            pltpu.VMEM((4, 8, 64), jnp.int32),
            pltpu.VMEM((4, 8, 64), jnp.int32),
            pltpu.VMEM((4, 64, F), jnp.float32),
            pltpu.SemaphoreType.DMA((4,)),
            pltpu.SemaphoreType.DMA((4,)),
            pltpu.SemaphoreType.DMA((4,)),
        ],
    )(g, row2d, colm2d)


BR = 512


def _tc_prep_body(degp_ref, xp_ref, dinv_ref, g0_ref):
    deg = degp_ref[0, :] + degp_ref[1, :]
    dv = jnp.where(deg > 0, lax.rsqrt(jnp.maximum(deg, 1.0)), 0.0)
    dinv_ref[:, 0] = dv
    g0_ref[...] = xp_ref[...] * dv[:, None]


def _tc_prep(degp, xp):
    return pl.pallas_call(
        _tc_prep_body,
        grid=(NP // BR,),
        in_specs=[pl.BlockSpec((NC, BR), lambda i: (0, i)),
                  pl.BlockSpec((BR, F), lambda i: (i, 0))],
        out_specs=[pl.BlockSpec((BR, 1), lambda i: (i, 0)),
                   pl.BlockSpec((BR, F), lambda i: (i, 0))],
        out_shape=[jax.ShapeDtypeStruct((NP, 1), jnp.float32),
                   jax.ShapeDtypeStruct((NP, F), jnp.float32)],
    )(degp, xp)


def _tc_combine_body(s_ref, dinv_ref, t1_ref, g1_ref):
    s = s_ref[0] + s_ref[1]
    dv = dinv_ref[...]
    t = (-dv) * s
    t1_ref[...] = t
    g1_ref[...] = dv * t


def _tc_combine(s, dinv):
    return pl.pallas_call(
        _tc_combine_body,
        grid=(NP // BR,),
        in_specs=[pl.BlockSpec((NC, BR, F), lambda i: (0, i, 0)),
                  pl.BlockSpec((BR, 1), lambda i: (i, 0))],
        out_specs=[pl.BlockSpec((BR, F), lambda i: (i, 0)),
                   pl.BlockSpec((BR, F), lambda i: (i, 0))],
        out_shape=[jax.ShapeDtypeStruct((NP, F), jnp.float32),
                   jax.ShapeDtypeStruct((NP, F), jnp.float32)],
    )(s, dinv)


def _tc_layer_body(h_ref, t1_ref, s2_ref, dinv_ref, w0_ref, w1_ref, w2_ref,
                   b_ref, hn_ref, gn_ref, *, relu):
    h = h_ref[...]
    dv = dinv_ref[...]
    t2 = (-2.0 * dv) * (s2_ref[0] + s2_ref[1]) - h
    z = jnp.dot(h, w0_ref[...], preferred_element_type=jnp.float32)
    z += jnp.dot(t1_ref[...], w1_ref[...], preferred_element_type=jnp.float32)
    z += jnp.dot(t2, w2_ref[...], preferred_element_type=jnp.float32)
    z += b_ref[...]
    if relu:
        z = jnp.clip(z, 0.0, 6.0)
    hn_ref[...] = z
    if gn_ref is not None:
        gn_ref[...] = dv * z


def _tc_layer(h, t1, s2, dinv, w, b, relu, want_g):
    nouts = 2 if want_g else 1
    body = functools.partial(_tc_layer_body, relu=relu)
    if not want_g:
        def body(h_ref, t1_ref, s2_ref, dinv_ref, w0, w1, w2, b_ref, hn_ref):
            _tc_layer_body(h_ref, t1_ref, s2_ref, dinv_ref, w0, w1, w2,
                           b_ref, hn_ref, None, relu=relu)
    outs = pl.pallas_call(
        body,
        grid=(NP // BR,),
        in_specs=[pl.BlockSpec((BR, F), lambda i: (i, 0)),
                  pl.BlockSpec((BR, F), lambda i: (i, 0)),
                  pl.BlockSpec((NC, BR, F), lambda i: (0, i, 0)),
                  pl.BlockSpec((BR, 1), lambda i: (i, 0)),
                  pl.BlockSpec((F, F), lambda i: (0, 0)),
                  pl.BlockSpec((F, F), lambda i: (0, 0)),
                  pl.BlockSpec((F, F), lambda i: (0, 0)),
                  pl.BlockSpec((1, F), lambda i: (0, 0))],
        out_specs=[pl.BlockSpec((BR, F), lambda i: (i, 0))] * nouts,
        out_shape=[jax.ShapeDtypeStruct((NP, F), jnp.float32)] * nouts,
    )(h, t1, s2, dinv, w[0], w[1], w[2], b.reshape(1, F))
    return outs if want_g else (outs[0] if isinstance(outs, (list, tuple)) else outs)


def kernel(x, edge_index, W1, b1, W2, b2, W3, b3, W4, b4):
    row2d = edge_index[0].astype(jnp.int32).reshape(EB, 128)
    col2d = edge_index[1].astype(jnp.int32).reshape(EB, 128)
    row2d = jnp.pad(row2d, ((0, EBP - EB), (0, 0)))
    col2d = jnp.pad(col2d, ((0, EBP - EB), (0, 0)))
    xp = jnp.zeros((NP, F), x.dtype).at[:N].set(x)

    degp, colm2d = _sc_prep(row2d, col2d)
    dinv, g = _tc_prep(degp, xp)
    row64 = row2d.reshape(EBP2, 64)
    colm64 = colm2d.reshape(EBP2, 64)

    h = xp
    layers = [(W1, b1, True), (W2, b2, True), (W3, b3, True)]
    for (w, b, relu) in layers:
        s1 = _sc_prop(g, row64, colm64)
        t1, g1 = _tc_combine(s1, dinv)
        s2 = _sc_prop(g1, row64, colm64)
        h, g = _tc_layer(h, t1, s2, dinv, w, b, relu, True)

    w4p = jnp.zeros((3, F, F), jnp.float32).at[:, :, :W4.shape[2]].set(W4)
    b4p = jnp.zeros((F,), jnp.float32).at[:W4.shape[2]].set(b4)
    s1 = _sc_prop(g, row64, colm64)
    t1, g1 = _tc_combine(s1, dinv)
    s2 = _sc_prop(g1, row64, colm64)
    out = _tc_layer(h, t1, s2, dinv, w4p, b4p, False, False)
    return (out[:N, :W4.shape[2]], edge_index)


# speedup vs baseline: 15.8782x; 1.1534x over previous
import functools

import jax
import jax.numpy as jnp
from jax import lax
from jax.experimental import pallas as pl
from jax.experimental.pallas import tpu as pltpu
from jax.experimental.pallas import tpu_sc as plsc

N = 10000
E = 320000
F = 128
NP = 10240
EB = E // 128
EBP = 2560
EB2 = E // 64
EBP2 = 2 * EBP
NC, NS = 2, 16
NW = NC * NS
RPT = NP // NS

_MESH = dict(core_axis_name="c", subcore_axis_name="s", num_cores=NC,
             num_subcores=NS)


def _wid():
    return lax.axis_index("s") * NC + lax.axis_index("c")


def _blk_range(w):
    lo = ((w * EB) // NW) // 8 * 8
    hi = jnp.where(w == NW - 1, EB, (((w + 1) * EB) // NW) // 8 * 8)
    return lo, hi


def _blk_range2(w):
    lo = ((w * EB2) // NW) // 8 * 8
    hi = jnp.where(w == NW - 1, EB2, (((w + 1) * EB2) // NW) // 8 * 8)
    return lo, hi


def _sc_prep_body(row_hbm, col_hbm, degp_hbm, colm_hbm,
                  dacc, zbuf, ones_v, rowcb, colcb, rowmb, colmb, ssem):
    cc = lax.axis_index("c")
    ss = lax.axis_index("s")
    w = _wid()

    for j in range(8):
        ones_v[pl.ds(j * 16, 16)] = jnp.ones((16,), jnp.float32)

    def z_body(i, _):
        zbuf[pl.ds(i * 16, 16)] = jnp.zeros((16,), jnp.float32)
        return ()
    lax.fori_loop(0, RPT // 16, z_body, ())
    pltpu.sync_copy(zbuf, dacc.at[pl.ds(pl.multiple_of(ss * RPT, 8), RPT)])
    plsc.subcore_barrier()

    lo, hi = _blk_range(w)
    n = hi - lo
    nch = (n + 7) // 8
    trash = N + lax.iota(jnp.int32, 16)

    def chunk_body(c, _):
        off = pl.multiple_of(lo + c * 8, 8)
        pltpu.sync_copy(row_hbm.at[pl.ds(off, 8)], rowcb)
        pltpu.sync_copy(col_hbm.at[pl.ds(off, 8)], colcb)
        for q in range(8):
            for j in range(8):
                rv = rowcb[q, pl.ds(j * 16, 16)]
                cv = colcb[q, pl.ds(j * 16, 16)]
                eq = rv == cv
                rowmb[q, pl.ds(j * 16, 16)] = jnp.where(eq, trash, rv)
                colmb[q, pl.ds(j * 16, 16)] = jnp.where(eq, trash, cv)
            pltpu.async_copy(ones_v, dacc.at[rowmb.at[q]], ssem, add=True)
        pltpu.sync_copy(colmb, colm_hbm.at[pl.ds(off, 8)])
        for q in range(8):
            pltpu.make_async_copy(
                ones_v, dacc.at[rowmb.at[q]], ssem).wait()
        return ()
    lax.fori_loop(0, nch, chunk_body, ())

    plsc.subcore_barrier()
    dof = pl.multiple_of(ss * RPT, 8)
    pltpu.sync_copy(dacc.at[pl.ds(dof, RPT)], degp_hbm.at[cc, pl.ds(dof, RPT)])


def _sc_prep(row2d, col2d):
    return pl.kernel(
        _sc_prep_body,
        out_type=[jax.ShapeDtypeStruct((NC, NP), jnp.float32),
                  jax.ShapeDtypeStruct((EBP, 128), jnp.int32)],
        mesh=plsc.VectorSubcoreMesh(**_MESH),
        scratch_types=[
            pltpu.VMEM_SHARED((NP,), jnp.float32),
            pltpu.VMEM((RPT,), jnp.float32),
            pltpu.VMEM((128,), jnp.float32),
            pltpu.VMEM((8, 128), jnp.int32),
            pltpu.VMEM((8, 128), jnp.int32),
            pltpu.VMEM((8, 128), jnp.int32),
            pltpu.VMEM((8, 128), jnp.int32),
            pltpu.SemaphoreType.DMA,
        ],
        compiler_params=pltpu.CompilerParams(needs_layout_passes=False),
    )(row2d, col2d)


def _sc_prop_body(g_hbm, row_hbm, colm_hbm, s_hbm,
                  acc, rowvb, colvb, rows, gsem, ssem, csem):
    cc = lax.axis_index("c")
    ss = lax.axis_index("s")
    w = _wid()

    lo, hi = _blk_range2(w)
    n = hi - lo

    def _idx(k):
        return lax.rem(k, 5), lax.rem(k, 8), lax.rem(k // 8, 4)

    def _gather(k):
        kb, r, cp = _idx(k)
        return g_hbm.at[rowvb.at[cp, r]], rows.at[kb], gsem.at[kb]

    def _scatter(k):
        kb, r, cp = _idx(k)
        return rows.at[kb], acc.at[colvb.at[cp, r]], ssem.at[kb]

    def _load_chunk(base, slot):
        off = pl.multiple_of(base, 8)
        pltpu.async_copy(row_hbm.at[pl.ds(off, 8)], rowvb.at[slot],
                         csem.at[slot])
        pltpu.async_copy(colm_hbm.at[pl.ds(off, 8)], colvb.at[slot],
                         csem.at[slot])

    def _wait_chunk(base, slot):
        off = pl.multiple_of(base, 8)
        pltpu.make_async_copy(row_hbm.at[pl.ds(off, 8)], rowvb.at[slot],
                              csem.at[slot]).wait()
        pltpu.make_async_copy(colm_hbm.at[pl.ds(off, 8)], colvb.at[slot],
                              csem.at[slot]).wait()

    def zb_body(i, _):
        for j in range(8):
            rows[0, i, pl.ds(j * 16, 16)] = jnp.zeros((16,), jnp.float32)
        return ()
    lax.fori_loop(0, 64, zb_body, ())

    def zc_body(t, _):
        pltpu.sync_copy(
            rows.at[0],
            acc.at[pl.ds(pl.multiple_of(ss * RPT + t * 64, 8), 64)])
        return ()
    lax.fori_loop(0, RPT // 64, zc_body, ())

    lo8 = pl.multiple_of(lo, 8)
    _load_chunk(lo8, 0)
    _load_chunk(lo8 + 8, 1)
    _wait_chunk(lo8, 0)
    for k0 in range(3):
        pltpu.async_copy(*_gather(k0))
    plsc.subcore_barrier()

    def body(k, _):
        @pl.when(k >= 2)
        def _():
            s, d, m = _scatter(k - 2)
            pltpu.make_async_copy(s, d, m).wait()

        @pl.when(lax.rem(k, 8) == 0)
        def _():
            j = k // 8
            _wait_chunk(lo8 + (j + 1) * 8, lax.rem(j + 1, 4))
            _load_chunk(lo8 + (j + 2) * 8, lax.rem(j + 2, 4))

        s, d, m = _gather(k)
        pltpu.make_async_copy(s, d, m).wait()
        pltpu.async_copy(*_scatter(k), add=True)

        @pl.when(k + 3 < n)
        def _():
            pltpu.async_copy(*_gather(k + 3))
        return ()
    lax.fori_loop(0, n, body, ())

    for tail in range(2):
        k = n - 2 + tail
        s, d, m = _scatter(k)
        pltpu.make_async_copy(s, d, m).wait()
    jl = (n - 1) // 8 + 2
    _wait_chunk(lo8 + jl * 8, lax.rem(jl, 4))

    plsc.subcore_barrier()
    dof = pl.multiple_of(ss * RPT, 8)
    pltpu.sync_copy(acc.at[pl.ds(dof, RPT)], s_hbm.at[cc, pl.ds(dof, RPT)])


def _sc_prop(g, row2d, colm2d):
    return pl.kernel(
        _sc_prop_body,
        out_type=jax.ShapeDtypeStruct((NC, NP, F), jnp.float32),
        mesh=plsc.VectorSubcoreMesh(**_MESH),
        scratch_types=[
            pltpu.VMEM_SHARED((NP, F), jnp.float32),
            pltpu.VMEM((4, 8, 64), jnp.int32),
            pltpu.VMEM((4, 8, 64), jnp.int32),
            pltpu.VMEM((5, 64, F), jnp.float32),
            pltpu.SemaphoreType.DMA((5,)),
            pltpu.SemaphoreType.DMA((5,)),
            pltpu.SemaphoreType.DMA((4,)),
        ],
    )(g, row2d, colm2d)


BR = 512


def _tc_prep_body(degp_ref, xp_ref, dinv_ref, g0_ref):
    deg = degp_ref[0, :] + degp_ref[1, :]
    dv = jnp.where(deg > 0, lax.rsqrt(jnp.maximum(deg, 1.0)), 0.0)
    dinv_ref[:, 0] = dv
    g0_ref[...] = xp_ref[...] * dv[:, None]


def _tc_prep(degp, xp):
    return pl.pallas_call(
        _tc_prep_body,
        grid=(NP // BR,),
        in_specs=[pl.BlockSpec((NC, BR), lambda i: (0, i)),
                  pl.BlockSpec((BR, F), lambda i: (i, 0))],
        out_specs=[pl.BlockSpec((BR, 1), lambda i: (i, 0)),
                   pl.BlockSpec((BR, F), lambda i: (i, 0))],
        out_shape=[jax.ShapeDtypeStruct((NP, 1), jnp.float32),
                   jax.ShapeDtypeStruct((NP, F), jnp.float32)],
    )(degp, xp)


def _tc_combine_body(s_ref, dinv_ref, t1_ref, g1_ref):
    s = s_ref[0] + s_ref[1]
    dv = dinv_ref[...]
    t = (-dv) * s
    t1_ref[...] = t
    g1_ref[...] = dv * t


def _tc_combine(s, dinv):
    return pl.pallas_call(
        _tc_combine_body,
        grid=(NP // BR,),
        in_specs=[pl.BlockSpec((NC, BR, F), lambda i: (0, i, 0)),
                  pl.BlockSpec((BR, 1), lambda i: (i, 0))],
        out_specs=[pl.BlockSpec((BR, F), lambda i: (i, 0)),
                   pl.BlockSpec((BR, F), lambda i: (i, 0))],
        out_shape=[jax.ShapeDtypeStruct((NP, F), jnp.float32),
                   jax.ShapeDtypeStruct((NP, F), jnp.float32)],
    )(s, dinv)


def _tc_layer_body(h_ref, t1_ref, s2_ref, dinv_ref, w0_ref, w1_ref, w2_ref,
                   b_ref, hn_ref, gn_ref, *, relu):
    h = h_ref[...]
    dv = dinv_ref[...]
    t2 = (-2.0 * dv) * (s2_ref[0] + s2_ref[1]) - h
    z = jnp.dot(h, w0_ref[...], preferred_element_type=jnp.float32)
    z += jnp.dot(t1_ref[...], w1_ref[...], preferred_element_type=jnp.float32)
    z += jnp.dot(t2, w2_ref[...], preferred_element_type=jnp.float32)
    z += b_ref[...]
    if relu:
        z = jnp.clip(z, 0.0, 6.0)
    hn_ref[...] = z
    if gn_ref is not None:
        gn_ref[...] = dv * z


def _tc_layer(h, t1, s2, dinv, w, b, relu, want_g):
    nouts = 2 if want_g else 1
    body = functools.partial(_tc_layer_body, relu=relu)
    if not want_g:
        def body(h_ref, t1_ref, s2_ref, dinv_ref, w0, w1, w2, b_ref, hn_ref):
            _tc_layer_body(h_ref, t1_ref, s2_ref, dinv_ref, w0, w1, w2,
                           b_ref, hn_ref, None, relu=relu)
    outs = pl.pallas_call(
        body,
        grid=(NP // BR,),
        in_specs=[pl.BlockSpec((BR, F), lambda i: (i, 0)),
                  pl.BlockSpec((BR, F), lambda i: (i, 0)),
                  pl.BlockSpec((NC, BR, F), lambda i: (0, i, 0)),
                  pl.BlockSpec((BR, 1), lambda i: (i, 0)),
                  pl.BlockSpec((F, F), lambda i: (0, 0)),
                  pl.BlockSpec((F, F), lambda i: (0, 0)),
                  pl.BlockSpec((F, F), lambda i: (0, 0)),
                  pl.BlockSpec((1, F), lambda i: (0, 0))],
        out_specs=[pl.BlockSpec((BR, F), lambda i: (i, 0))] * nouts,
        out_shape=[jax.ShapeDtypeStruct((NP, F), jnp.float32)] * nouts,
    )(h, t1, s2, dinv, w[0], w[1], w[2], b.reshape(1, F))
    return outs if want_g else (outs[0] if isinstance(outs, (list, tuple)) else outs)


def kernel(x, edge_index, W1, b1, W2, b2, W3, b3, W4, b4):
    row2d = edge_index[0].astype(jnp.int32).reshape(EB, 128)
    col2d = edge_index[1].astype(jnp.int32).reshape(EB, 128)
    row2d = jnp.pad(row2d, ((0, EBP - EB), (0, 0)))
    col2d = jnp.pad(col2d, ((0, EBP - EB), (0, 0)))
    xp = jnp.zeros((NP, F), x.dtype).at[:N].set(x)

    degp, colm2d = _sc_prep(row2d, col2d)
    dinv, g = _tc_prep(degp, xp)
    row64 = row2d.reshape(EBP2, 64)
    colm64 = colm2d.reshape(EBP2, 64)

    h = xp
    layers = [(W1, b1, True), (W2, b2, True), (W3, b3, True)]
    for (w, b, relu) in layers:
        s1 = _sc_prop(g, row64, colm64)
        t1, g1 = _tc_combine(s1, dinv)
        s2 = _sc_prop(g1, row64, colm64)
        h, g = _tc_layer(h, t1, s2, dinv, w, b, relu, True)

    w4p = jnp.zeros((3, F, F), jnp.float32).at[:, :, :W4.shape[2]].set(W4)
    b4p = jnp.zeros((F,), jnp.float32).at[:W4.shape[2]].set(b4)
    s1 = _sc_prop(g, row64, colm64)
    t1, g1 = _tc_combine(s1, dinv)
    s2 = _sc_prop(g1, row64, colm64)
    out = _tc_layer(h, t1, s2, dinv, w4p, b4p, False, False)
    return (out[:N, :W4.shape[2]], edge_index)


# speedup vs baseline: 16.9884x; 1.0699x over previous
import functools

import jax
import jax.numpy as jnp
from jax import lax
from jax.experimental import pallas as pl
from jax.experimental.pallas import tpu as pltpu
from jax.experimental.pallas import tpu_sc as plsc

N = 10000
E = 320000
F = 128
NP = 10240
EB = E // 128
EBP = 2560
EB2 = E // 64
EBP2 = 2 * EBP
NC, NS = 2, 16
NW = NC * NS
RPT = NP // NS

_MESH = dict(core_axis_name="c", subcore_axis_name="s", num_cores=NC,
             num_subcores=NS)


def _wid():
    return lax.axis_index("s") * NC + lax.axis_index("c")


def _blk_range(w):
    lo = ((w * EB) // NW) // 8 * 8
    hi = jnp.where(w == NW - 1, EB, (((w + 1) * EB) // NW) // 8 * 8)
    return lo, hi


def _blk_range2(w):
    lo = ((w * EB2) // NW) // 8 * 8
    hi = jnp.where(w == NW - 1, EB2, (((w + 1) * EB2) // NW) // 8 * 8)
    return lo, hi


def _sc_prep_body(row_hbm, col_hbm, degp_hbm, colm_hbm,
                  dacc, zbuf, ones_v, rowcb, colcb, rowmb, colmb, ssem):
    cc = lax.axis_index("c")
    ss = lax.axis_index("s")
    w = _wid()

    for j in range(8):
        ones_v[pl.ds(j * 16, 16)] = jnp.ones((16,), jnp.float32)

    def z_body(i, _):
        zbuf[pl.ds(i * 16, 16)] = jnp.zeros((16,), jnp.float32)
        return ()
    lax.fori_loop(0, RPT // 16, z_body, ())
    pltpu.sync_copy(zbuf, dacc.at[pl.ds(pl.multiple_of(ss * RPT, 8), RPT)])
    plsc.subcore_barrier()

    lo, hi = _blk_range(w)
    n = hi - lo
    nch = (n + 7) // 8
    trash = N + lax.iota(jnp.int32, 16)

    def chunk_body(c, _):
        off = pl.multiple_of(lo + c * 8, 8)
        pltpu.sync_copy(row_hbm.at[pl.ds(off, 8)], rowcb)
        pltpu.sync_copy(col_hbm.at[pl.ds(off, 8)], colcb)
        for q in range(8):
            for j in range(8):
                rv = rowcb[q, pl.ds(j * 16, 16)]
                cv = colcb[q, pl.ds(j * 16, 16)]
                eq = rv == cv
                rowmb[q, pl.ds(j * 16, 16)] = jnp.where(eq, trash, rv)
                colmb[q, pl.ds(j * 16, 16)] = jnp.where(eq, trash, cv)
            pltpu.async_copy(ones_v, dacc.at[rowmb.at[q]], ssem, add=True)
        pltpu.sync_copy(colmb, colm_hbm.at[pl.ds(off, 8)])
        for q in range(8):
            pltpu.make_async_copy(
                ones_v, dacc.at[rowmb.at[q]], ssem).wait()
        return ()
    lax.fori_loop(0, nch, chunk_body, ())

    plsc.subcore_barrier()
    dof = pl.multiple_of(ss * RPT, 8)
    pltpu.sync_copy(dacc.at[pl.ds(dof, RPT)], degp_hbm.at[cc, pl.ds(dof, RPT)])


def _sc_prep(row2d, col2d):
    return pl.kernel(
        _sc_prep_body,
        out_type=[jax.ShapeDtypeStruct((NC, NP), jnp.float32),
                  jax.ShapeDtypeStruct((EBP, 128), jnp.int32)],
        mesh=plsc.VectorSubcoreMesh(**_MESH),
        scratch_types=[
            pltpu.VMEM_SHARED((NP,), jnp.float32),
            pltpu.VMEM((RPT,), jnp.float32),
            pltpu.VMEM((128,), jnp.float32),
            pltpu.VMEM((8, 128), jnp.int32),
            pltpu.VMEM((8, 128), jnp.int32),
            pltpu.VMEM((8, 128), jnp.int32),
            pltpu.VMEM((8, 128), jnp.int32),
            pltpu.SemaphoreType.DMA,
        ],
        compiler_params=pltpu.CompilerParams(needs_layout_passes=False),
    )(row2d, col2d)


def _sc_prop_body(g_hbm, row_hbm, colm_hbm, s_hbm,
                  acc, rowvb, colvb, rows, gsem, ssem, csem):
    cc = lax.axis_index("c")
    ss = lax.axis_index("s")
    w = _wid()

    lo, hi = _blk_range2(w)
    n = hi - lo

    def _idx(k):
        return lax.rem(k, 5), lax.rem(k, 8), lax.rem(k // 8, 4)

    def _gather(k):
        kb, r, cp = _idx(k)
        return g_hbm.at[rowvb.at[cp, r]], rows.at[kb], gsem.at[kb]

    def _scatter(k):
        kb, r, cp = _idx(k)
        return rows.at[kb], acc.at[colvb.at[cp, r]], ssem.at[kb]

    def _load_chunk(base, slot):
        off = pl.multiple_of(base, 8)
        pltpu.async_copy(row_hbm.at[pl.ds(off, 8)], rowvb.at[slot],
                         csem.at[slot])
        pltpu.async_copy(colm_hbm.at[pl.ds(off, 8)], colvb.at[slot],
                         csem.at[slot])

    def _wait_chunk(base, slot):
        off = pl.multiple_of(base, 8)
        pltpu.make_async_copy(row_hbm.at[pl.ds(off, 8)], rowvb.at[slot],
                              csem.at[slot]).wait()
        pltpu.make_async_copy(colm_hbm.at[pl.ds(off, 8)], colvb.at[slot],
                              csem.at[slot]).wait()

    def zb_body(i, _):
        for j in range(8):
            rows[0, i, pl.ds(j * 16, 16)] = jnp.zeros((16,), jnp.float32)
        return ()
    lax.fori_loop(0, 64, zb_body, ())

    def zc_body(t, _):
        pltpu.sync_copy(
            rows.at[0],
            acc.at[pl.ds(pl.multiple_of(ss * RPT + t * 64, 8), 64)])
        return ()
    lax.fori_loop(0, RPT // 64, zc_body, ())

    lo8 = pl.multiple_of(lo, 8)
    _load_chunk(lo8, 0)
    _load_chunk(lo8 + 8, 1)
    _wait_chunk(lo8, 0)
    for k0 in range(4):
        pltpu.async_copy(*_gather(k0))
    plsc.subcore_barrier()

    def body(k, _):
        @pl.when(k >= 1)
        def _():
            s, d, m = _scatter(k - 1)
            pltpu.make_async_copy(s, d, m).wait()

        @pl.when(lax.rem(k, 8) == 0)
        def _():
            j = k // 8
            _wait_chunk(lo8 + (j + 1) * 8, lax.rem(j + 1, 4))
            _load_chunk(lo8 + (j + 2) * 8, lax.rem(j + 2, 4))

        s, d, m = _gather(k)
        pltpu.make_async_copy(s, d, m).wait()
        pltpu.async_copy(*_scatter(k), add=True)

        @pl.when(k + 4 < n)
        def _():
            pltpu.async_copy(*_gather(k + 4))
        return ()
    lax.fori_loop(0, n, body, ())

    s, d, m = _scatter(n - 1)
    pltpu.make_async_copy(s, d, m).wait()
    jl = (n - 1) // 8 + 2
    _wait_chunk(lo8 + jl * 8, lax.rem(jl, 4))

    plsc.subcore_barrier()
    dof = pl.multiple_of(ss * RPT, 8)
    pltpu.sync_copy(acc.at[pl.ds(dof, RPT)], s_hbm.at[cc, pl.ds(dof, RPT)])


def _sc_prop(g, row2d, colm2d):
    return pl.kernel(
        _sc_prop_body,
        out_type=jax.ShapeDtypeStruct((NC, NP, F), jnp.float32),
        mesh=plsc.VectorSubcoreMesh(**_MESH),
        scratch_types=[
            pltpu.VMEM_SHARED((NP, F), jnp.float32),
            pltpu.VMEM((4, 8, 64), jnp.int32),
            pltpu.VMEM((4, 8, 64), jnp.int32),
            pltpu.VMEM((5, 64, F), jnp.float32),
            pltpu.SemaphoreType.DMA((5,)),
            pltpu.SemaphoreType.DMA((5,)),
            pltpu.SemaphoreType.DMA((4,)),
        ],
    )(g, row2d, colm2d)


BR = 512


def _tc_prep_body(degp_ref, xp_ref, dinv_ref, g0_ref):
    deg = degp_ref[0, :] + degp_ref[1, :]
    dv = jnp.where(deg > 0, lax.rsqrt(jnp.maximum(deg, 1.0)), 0.0)
    dinv_ref[:, 0] = dv
    g0_ref[...] = xp_ref[...] * dv[:, None]


def _tc_prep(degp, xp):
    return pl.pallas_call(
        _tc_prep_body,
        grid=(NP // BR,),
        in_specs=[pl.BlockSpec((NC, BR), lambda i: (0, i)),
                  pl.BlockSpec((BR, F), lambda i: (i, 0))],
        out_specs=[pl.BlockSpec((BR, 1), lambda i: (i, 0)),
                   pl.BlockSpec((BR, F), lambda i: (i, 0))],
        out_shape=[jax.ShapeDtypeStruct((NP, 1), jnp.float32),
                   jax.ShapeDtypeStruct((NP, F), jnp.float32)],
    )(degp, xp)


def _tc_combine_body(s_ref, dinv_ref, t1_ref, g1_ref):
    s = s_ref[0] + s_ref[1]
    dv = dinv_ref[...]
    t = (-dv) * s
    t1_ref[...] = t
    g1_ref[...] = dv * t


def _tc_combine(s, dinv):
    return pl.pallas_call(
        _tc_combine_body,
        grid=(NP // BR,),
        in_specs=[pl.BlockSpec((NC, BR, F), lambda i: (0, i, 0)),
                  pl.BlockSpec((BR, 1), lambda i: (i, 0))],
        out_specs=[pl.BlockSpec((BR, F), lambda i: (i, 0)),
                   pl.BlockSpec((BR, F), lambda i: (i, 0))],
        out_shape=[jax.ShapeDtypeStruct((NP, F), jnp.float32),
                   jax.ShapeDtypeStruct((NP, F), jnp.float32)],
    )(s, dinv)


def _tc_layer_body(h_ref, t1_ref, s2_ref, dinv_ref, w0_ref, w1_ref, w2_ref,
                   b_ref, hn_ref, gn_ref, *, relu):
    h = h_ref[...]
    dv = dinv_ref[...]
    t2 = (-2.0 * dv) * (s2_ref[0] + s2_ref[1]) - h
    z = jnp.dot(h, w0_ref[...], preferred_element_type=jnp.float32)
    z += jnp.dot(t1_ref[...], w1_ref[...], preferred_element_type=jnp.float32)
    z += jnp.dot(t2, w2_ref[...], preferred_element_type=jnp.float32)
    z += b_ref[...]
    if relu:
        z = jnp.clip(z, 0.0, 6.0)
    hn_ref[...] = z
    if gn_ref is not None:
        gn_ref[...] = dv * z


def _tc_layer(h, t1, s2, dinv, w, b, relu, want_g):
    nouts = 2 if want_g else 1
    body = functools.partial(_tc_layer_body, relu=relu)
    if not want_g:
        def body(h_ref, t1_ref, s2_ref, dinv_ref, w0, w1, w2, b_ref, hn_ref):
            _tc_layer_body(h_ref, t1_ref, s2_ref, dinv_ref, w0, w1, w2,
                           b_ref, hn_ref, None, relu=relu)
    outs = pl.pallas_call(
        body,
        grid=(NP // BR,),
        in_specs=[pl.BlockSpec((BR, F), lambda i: (i, 0)),
                  pl.BlockSpec((BR, F), lambda i: (i, 0)),
                  pl.BlockSpec((NC, BR, F), lambda i: (0, i, 0)),
                  pl.BlockSpec((BR, 1), lambda i: (i, 0)),
                  pl.BlockSpec((F, F), lambda i: (0, 0)),
                  pl.BlockSpec((F, F), lambda i: (0, 0)),
                  pl.BlockSpec((F, F), lambda i: (0, 0)),
                  pl.BlockSpec((1, F), lambda i: (0, 0))],
        out_specs=[pl.BlockSpec((BR, F), lambda i: (i, 0))] * nouts,
        out_shape=[jax.ShapeDtypeStruct((NP, F), jnp.float32)] * nouts,
    )(h, t1, s2, dinv, w[0], w[1], w[2], b.reshape(1, F))
    return outs if want_g else (outs[0] if isinstance(outs, (list, tuple)) else outs)


def kernel(x, edge_index, W1, b1, W2, b2, W3, b3, W4, b4):
    row2d = edge_index[0].astype(jnp.int32).reshape(EB, 128)
    col2d = edge_index[1].astype(jnp.int32).reshape(EB, 128)
    row2d = jnp.pad(row2d, ((0, EBP - EB), (0, 0)))
    col2d = jnp.pad(col2d, ((0, EBP - EB), (0, 0)))
    xp = jnp.zeros((NP, F), x.dtype).at[:N].set(x)

    degp, colm2d = _sc_prep(row2d, col2d)
    dinv, g = _tc_prep(degp, xp)
    row64 = row2d.reshape(EBP2, 64)
    colm64 = colm2d.reshape(EBP2, 64)

    h = xp
    layers = [(W1, b1, True), (W2, b2, True), (W3, b3, True)]
    for (w, b, relu) in layers:
        s1 = _sc_prop(g, row64, colm64)
        t1, g1 = _tc_combine(s1, dinv)
        s2 = _sc_prop(g1, row64, colm64)
        h, g = _tc_layer(h, t1, s2, dinv, w, b, relu, True)

    w4p = jnp.zeros((3, F, F), jnp.float32).at[:, :, :W4.shape[2]].set(W4)
    b4p = jnp.zeros((F,), jnp.float32).at[:W4.shape[2]].set(b4)
    s1 = _sc_prop(g, row64, colm64)
    t1, g1 = _tc_combine(s1, dinv)
    s2 = _sc_prop(g1, row64, colm64)
    out = _tc_layer(h, t1, s2, dinv, w4p, b4p, False, False)
    return (out[:N, :W4.shape[2]], edge_index)
